# Initial kernel scaffold; baseline (speedup 1.0000x reference)
import jax
import jax.numpy as jnp
from jax.experimental import pallas as pl


def kernel(x, edge_index_cites, edge_index_refs, edge_index_likes, W_cites, W_refs, W_likes, gamma, beta):
    raise NotImplementedError("write your pallas kernel here")


# speedup vs baseline: 4.1820x; 4.1820x over previous
import functools

import jax
import jax.numpy as jnp
from jax import lax
from jax.experimental import pallas as pl
from jax.experimental.pallas import tpu as pltpu
from jax.experimental.pallas import tpu_sc as plsc

N = 10000
D = 256
HALF = 128
E = 160000
NR = 3
NCSC = 2
NS = 16
CH = 80
PT = E // NS
NCHUNK = PT // CH
NPAD = 10240
ROWS_T = NPAD // NS
DR = 128
BN_EPS = 1e-5
L2_EPS = 1e-12

_MESH = dict(core_axis_name="c", subcore_axis_name="s", num_cores=NCSC,
             num_subcores=NS)


def _deg_body(idx6, zeros1, ones_c, deg_out, h0, h1, h2, ibuf, obuf, zbuf,
              hbuf):
    c = lax.axis_index("c")
    s = lax.axis_index("s")
    hists = (h0, h1, h2)
    pltpu.sync_copy(ones_c, obuf)

    @pl.when(s < 3)
    def _():
        pltpu.sync_copy(zeros1, zbuf)

    def _zero(j):
        @pl.when(s == j)
        def _():
            pltpu.sync_copy(zbuf, hists[j])

    for j in range(NR):
        _zero(j)
    plsc.subcore_barrier()

    for j in range(NR):
        a = c * NR + j
        pltpu.sync_copy(idx6.at[a, s], ibuf)

        def _chunk(ci, carry, _hist=hists[j]):
            pltpu.sync_copy(obuf, _hist.at[ibuf.at[ci]], add=True)
            return carry

        lax.fori_loop(0, NCHUNK, _chunk, 0)
    plsc.subcore_barrier()

    def _drain(j):
        @pl.when(s == j)
        def _():
            pltpu.sync_copy(hists[j], hbuf.at[0])
            pltpu.sync_copy(hbuf, deg_out.at[c * NR + j])

    for j in range(NR):
        _drain(j)


def _degrees(idx6, zeros1, ones_c):
    return pl.kernel(
        _deg_body,
        out_type=jax.ShapeDtypeStruct((2 * NR, 1, N), jnp.float32),
        mesh=plsc.VectorSubcoreMesh(**_MESH),
        scratch_types=[
            pltpu.VMEM_SHARED((N,), jnp.float32),
            pltpu.VMEM_SHARED((N,), jnp.float32),
            pltpu.VMEM_SHARED((N,), jnp.float32),
            pltpu.VMEM((NCHUNK, CH), jnp.int32),
            pltpu.VMEM((CH,), jnp.float32),
            pltpu.VMEM((N,), jnp.float32),
            pltpu.VMEM((1, N), jnp.float32),
        ],
    )(idx6, zeros1, ones_c)


def _mm_body(x_ref, n_ref, w_ref, o_ref):
    xs = x_ref[...] * n_ref[0]
    o_ref[0, 0] = jnp.dot(xs, w_ref[0], preferred_element_type=jnp.float32)


def _tables(x, nsrc_col, W3, bn):
    nb = N // bn
    return pl.pallas_call(
        _mm_body,
        grid=(NCSC, NR, nb),
        in_specs=[
            pl.BlockSpec((bn, D), lambda c, r, b: (b, 0)),
            pl.BlockSpec((1, bn, 1), lambda c, r, b: (r, b, 0)),
            pl.BlockSpec((1, D, HALF), lambda c, r, b: (r, 0, c)),
        ],
        out_specs=pl.BlockSpec((1, 1, bn, HALF), lambda c, r, b: (c, r, b, 0)),
        out_shape=jax.ShapeDtypeStruct((NCSC, NR, N, HALF), jnp.float32),
    )(x, nsrc_col, W3)


def _scat_body(t0, t1, srcg, dsti, zrow, out, acc, sbuf, dbuf, buf, sem):
    c = lax.axis_index("c")
    s = lax.axis_index("s")
    for r in range(NR):
        pltpu.sync_copy(zrow, buf)
        for k in range(ROWS_T // DR):
            pltpu.sync_copy(buf, acc.at[pl.ds(s * ROWS_T + k * DR, DR)])
        plsc.subcore_barrier()
        pltpu.sync_copy(srcg.at[r, s], sbuf)
        pltpu.sync_copy(dsti.at[r, s], dbuf)
        gbuf = buf.at[pl.ds(0, CH)]

        def _chunk(ci, carry):
            @pl.when(c == 0)
            def _():
                pltpu.async_copy(t0.at[sbuf.at[ci]], gbuf, sem).wait()

            @pl.when(c == 1)
            def _():
                pltpu.async_copy(t1.at[sbuf.at[ci]], gbuf, sem).wait()

            pltpu.sync_copy(gbuf, acc.at[dbuf.at[ci]], add=True)
            return carry

        lax.fori_loop(0, NCHUNK, _chunk, 0)
        plsc.subcore_barrier()
        for k in range(ROWS_T // DR):
            pltpu.sync_copy(acc.at[pl.ds(s * ROWS_T + k * DR, DR)], buf)
            pltpu.sync_copy(
                buf, out.at[r, c, pl.ds(s * ROWS_T + k * DR, DR)])


def _scatter(t0, t1, srcg, dsti, zrow):
    return pl.kernel(
        _scat_body,
        out_type=jax.ShapeDtypeStruct((NR, NCSC, NPAD, HALF), jnp.float32),
        mesh=plsc.VectorSubcoreMesh(**_MESH),
        scratch_types=[
            pltpu.VMEM_SHARED((NPAD, HALF), jnp.float32),
            pltpu.VMEM((NCHUNK, CH), jnp.int32),
            pltpu.VMEM((NCHUNK, CH), jnp.int32),
            pltpu.VMEM((DR, HALF), jnp.float32),
            pltpu.SemaphoreType.DMA,
        ],
    )(t0, t1, srcg, dsti, zrow)


def _comb_body(agg_ref, nd_ref, y_ref, st_ref):
    b = pl.program_id(0)
    n0 = nd_ref[0]
    n1 = nd_ref[1]
    n2 = nd_ref[2]
    y0 = agg_ref[0, 0] * n0 + agg_ref[1, 0] * n1 + agg_ref[2, 0] * n2
    y1 = agg_ref[0, 1] * n0 + agg_ref[1, 1] * n1 + agg_ref[2, 1] * n2
    yb = jnp.concatenate([y0, y1], axis=1)
    y_ref[...] = yb
    s1 = jnp.sum(yb, axis=0, keepdims=True)
    s2 = jnp.sum(yb * yb, axis=0, keepdims=True)
    upd = jnp.concatenate(
        [s1, s2, jnp.zeros((6, D), jnp.float32)], axis=0)

    @pl.when(b == 0)
    def _():
        st_ref[...] = jnp.zeros_like(st_ref)

    st_ref[...] += upd


def _combine(agg, nd_col, bn):
    nb = N // bn
    return pl.pallas_call(
        _comb_body,
        grid=(nb,),
        in_specs=[
            pl.BlockSpec((NR, NCSC, bn, HALF), lambda b: (0, 0, b, 0)),
            pl.BlockSpec((NR, bn, 1), lambda b: (0, b, 0)),
        ],
        out_specs=[
            pl.BlockSpec((bn, D), lambda b: (b, 0)),
            pl.BlockSpec((8, D), lambda b: (0, 0)),
        ],
        out_shape=[
            jax.ShapeDtypeStruct((N, D), jnp.float32),
            jax.ShapeDtypeStruct((8, D), jnp.float32),
        ],
    )(agg, nd_col)


def _norm_body(y_ref, sc_ref, sh_ref, o_ref):
    z = y_ref[...] * sc_ref[...] + sh_ref[...]
    nrm = jnp.sqrt(jnp.sum(z * z, axis=1, keepdims=True))
    o_ref[...] = z / jnp.maximum(nrm, L2_EPS)


def _bn_l2(y, scale, shift, bn):
    nb = N // bn
    return pl.pallas_call(
        _norm_body,
        grid=(nb,),
        in_specs=[
            pl.BlockSpec((bn, D), lambda b: (b, 0)),
            pl.BlockSpec((1, D), lambda b: (0, 0)),
            pl.BlockSpec((1, D), lambda b: (0, 0)),
        ],
        out_specs=pl.BlockSpec((bn, D), lambda b: (b, 0)),
        out_shape=jax.ShapeDtypeStruct((N, D), jnp.float32),
    )(y, scale, shift)


def kernel(x, edge_index_cites, edge_index_refs, edge_index_likes,
           W_cites, W_refs, W_likes, gamma, beta):
    srcs = jnp.stack(
        [edge_index_cites[0], edge_index_refs[0], edge_index_likes[0]])
    dsts = jnp.stack(
        [edge_index_cites[1], edge_index_refs[1], edge_index_likes[1]])

    idx6 = jnp.concatenate([srcs, dsts]).reshape(2 * NR, NS, NCHUNK, CH)
    zeros1 = jnp.zeros((N,), jnp.float32)
    ones_c = jnp.ones((CH,), jnp.float32)
    deg = _degrees(idx6, zeros1, ones_c)[:, 0, :]
    nsrc = lax.rsqrt(jnp.maximum(deg[:NR], 1.0))
    ndst = lax.rsqrt(jnp.maximum(deg[NR:], 1.0))

    W3 = jnp.stack([W_cites, W_refs, W_likes])
    table = _tables(x, nsrc[:, :, None], W3, 1000)
    t0 = table[0].reshape(NR * N, HALF)
    t1 = table[1].reshape(NR * N, HALF)

    roff = (jnp.arange(NR, dtype=jnp.int32) * N)[:, None]
    srcg = (srcs + roff).reshape(NR, NS, NCHUNK, CH)
    dsti = dsts.reshape(NR, NS, NCHUNK, CH)
    zrow = jnp.zeros((DR, HALF), jnp.float32)
    agg = _scatter(t0, t1, srcg, dsti, zrow)

    y, st = _combine(agg, ndst[:, :, None], 1000)
    mean = st[0] / N
    var = st[1] / N - mean * mean
    scale = gamma * lax.rsqrt(var + BN_EPS)
    shift = beta - mean * scale

    return _bn_l2(y, scale[None], shift[None], 1000)


# speedup vs baseline: 5.4261x; 1.2975x over previous
import functools

import jax
import jax.numpy as jnp
from jax import lax
from jax.experimental import pallas as pl
from jax.experimental.pallas import tpu as pltpu
from jax.experimental.pallas import tpu_sc as plsc

N = 10000
D = 256
HALF = 128
E = 160000
NR = 3
NCSC = 2
NS = 16
CH = 64
CHD = 80
PT = E // NS
PT2 = 10048
NCHUNK = PT2 // CH
NCHD = PT // CHD
NPAD = 10240
ROWS_T = NPAD // NS
DR = 64
BN_EPS = 1e-5
L2_EPS = 1e-12

_MESH = dict(core_axis_name="c", subcore_axis_name="s", num_cores=NCSC,
             num_subcores=NS)


def _deg_body(idx6, zeros1, ones_c, deg_out, h0, h1, h2, ibuf, obuf, zbuf,
              hbuf):
    c = lax.axis_index("c")
    s = lax.axis_index("s")
    hists = (h0, h1, h2)
    pltpu.sync_copy(ones_c, obuf)

    @pl.when(s < 3)
    def _():
        pltpu.sync_copy(zeros1, zbuf)

    def _zero(j):
        @pl.when(s == j)
        def _():
            pltpu.sync_copy(zbuf, hists[j])

    for j in range(NR):
        _zero(j)
    plsc.subcore_barrier()

    for j in range(NR):
        a = c * NR + j
        pltpu.sync_copy(idx6.at[a, s], ibuf)

        def _chunk(ci, carry, _hist=hists[j]):
            pltpu.sync_copy(obuf, _hist.at[ibuf.at[ci]], add=True)
            return carry

        lax.fori_loop(0, NCHD, _chunk, 0)
    plsc.subcore_barrier()

    def _drain(j):
        @pl.when(s == j)
        def _():
            pltpu.sync_copy(hists[j], hbuf.at[0])
            pltpu.sync_copy(hbuf, deg_out.at[c * NR + j])

    for j in range(NR):
        _drain(j)


def _degrees(idx6, zeros1, ones_c):
    return pl.kernel(
        _deg_body,
        out_type=jax.ShapeDtypeStruct((2 * NR, 1, N), jnp.float32),
        mesh=plsc.VectorSubcoreMesh(**_MESH),
        scratch_types=[
            pltpu.VMEM_SHARED((N,), jnp.float32),
            pltpu.VMEM_SHARED((N,), jnp.float32),
            pltpu.VMEM_SHARED((N,), jnp.float32),
            pltpu.VMEM((NCHD, CHD), jnp.int32),
            pltpu.VMEM((CHD,), jnp.float32),
            pltpu.VMEM((N,), jnp.float32),
            pltpu.VMEM((1, N), jnp.float32),
        ],
    )(idx6, zeros1, ones_c)


def _mm_body(x_ref, n_ref, w_ref, o_ref):
    xs = x_ref[...] * n_ref[0]
    o_ref[0, 0] = jnp.dot(xs, w_ref[0], preferred_element_type=jnp.float32)


def _tables(x, nsrc_col, W3, bn):
    nb = N // bn
    return pl.pallas_call(
        _mm_body,
        grid=(NCSC, NR, nb),
        in_specs=[
            pl.BlockSpec((bn, D), lambda c, r, b: (b, 0)),
            pl.BlockSpec((1, bn, 1), lambda c, r, b: (r, b, 0)),
            pl.BlockSpec((1, D, HALF), lambda c, r, b: (r, 0, c)),
        ],
        out_specs=pl.BlockSpec((1, 1, bn, HALF), lambda c, r, b: (c, r, b, 0)),
        out_shape=jax.ShapeDtypeStruct((NCSC, NR, N, HALF), jnp.float32),
    )(x, nsrc_col, W3)


def _scat_body(t0, t1, srcg, dsti, zrow, out, acc, sbuf, dbuf, bufs, gsem):
    c = lax.axis_index("c")
    s = lax.axis_index("s")

    def _gather_start(ci, slot):
        idx = sbuf.at[pl.ds(ci * CH, CH)]

        @pl.when(c == 0)
        def _():
            pltpu.async_copy(t0.at[idx], bufs.at[slot], gsem)

        @pl.when(c == 1)
        def _():
            pltpu.async_copy(t1.at[idx], bufs.at[slot], gsem)

    for r in range(NR):
        pltpu.sync_copy(zrow, bufs.at[0])
        for k in range(ROWS_T // DR):
            pltpu.sync_copy(bufs.at[0], acc.at[pl.ds(s * ROWS_T + k * DR, DR)])
        base = (r * NS + s) * PT2
        pltpu.sync_copy(srcg.at[pl.ds(base, PT2)], sbuf)
        pltpu.sync_copy(dsti.at[pl.ds(base, PT2)], dbuf)
        _gather_start(0, 0)
        plsc.subcore_barrier()

        def _chunk(ci, carry):
            slot = lax.rem(ci, 2)

            @pl.when(ci + 1 < NCHUNK)
            def _():
                _gather_start(ci + 1, lax.rem(ci + 1, 2))

            pltpu.make_async_copy(
                t0.at[sbuf.at[pl.ds(ci * CH, CH)]], bufs.at[slot],
                gsem).wait()
            for j in range(CH // 16):
                iv = dbuf[pl.ds(ci * CH + j * 16, 16)]
                pltpu.sync_copy(
                    bufs.at[slot, pl.ds(j * 16, 16)], acc.at[iv], add=True)
            return carry

        lax.fori_loop(0, NCHUNK, _chunk, 0)
        plsc.subcore_barrier()
        for k in range(ROWS_T // DR):
            pltpu.sync_copy(acc.at[pl.ds(s * ROWS_T + k * DR, DR)], bufs.at[0])
            pltpu.sync_copy(
                bufs.at[0], out.at[r, c, pl.ds(s * ROWS_T + k * DR, DR)])


def _scatter(t0, t1, srcg, dsti, zrow):
    return pl.kernel(
        _scat_body,
        out_type=jax.ShapeDtypeStruct((NR, NCSC, NPAD, HALF), jnp.float32),
        mesh=plsc.VectorSubcoreMesh(**_MESH),
        scratch_types=[
            pltpu.VMEM_SHARED((NPAD, HALF), jnp.float32),
            pltpu.VMEM((PT2,), jnp.int32),
            pltpu.VMEM((PT2,), jnp.int32),
            pltpu.VMEM((2, CH, HALF), jnp.float32),
            pltpu.SemaphoreType.DMA,
        ],
    )(t0, t1, srcg, dsti, zrow)


def _comb_body(agg_ref, nd_ref, y_ref, st_ref):
    b = pl.program_id(0)
    n0 = nd_ref[0]
    n1 = nd_ref[1]
    n2 = nd_ref[2]
    y0 = agg_ref[0, 0] * n0 + agg_ref[1, 0] * n1 + agg_ref[2, 0] * n2
    y1 = agg_ref[0, 1] * n0 + agg_ref[1, 1] * n1 + agg_ref[2, 1] * n2
    yb = jnp.concatenate([y0, y1], axis=1)
    y_ref[...] = yb
    s1 = jnp.sum(yb, axis=0, keepdims=True)
    s2 = jnp.sum(yb * yb, axis=0, keepdims=True)
    upd = jnp.concatenate(
        [s1, s2, jnp.zeros((6, D), jnp.float32)], axis=0)

    @pl.when(b == 0)
    def _():
        st_ref[...] = jnp.zeros_like(st_ref)

    st_ref[...] += upd


def _combine(agg, nd_col, bn):
    nb = N // bn
    return pl.pallas_call(
        _comb_body,
        grid=(nb,),
        in_specs=[
            pl.BlockSpec((NR, NCSC, bn, HALF), lambda b: (0, 0, b, 0)),
            pl.BlockSpec((NR, bn, 1), lambda b: (0, b, 0)),
        ],
        out_specs=[
            pl.BlockSpec((bn, D), lambda b: (b, 0)),
            pl.BlockSpec((8, D), lambda b: (0, 0)),
        ],
        out_shape=[
            jax.ShapeDtypeStruct((N, D), jnp.float32),
            jax.ShapeDtypeStruct((8, D), jnp.float32),
        ],
    )(agg, nd_col)


def _norm_body(y_ref, sc_ref, sh_ref, o_ref):
    z = y_ref[...] * sc_ref[...] + sh_ref[...]
    nrm = jnp.sqrt(jnp.sum(z * z, axis=1, keepdims=True))
    o_ref[...] = z / jnp.maximum(nrm, L2_EPS)


def _bn_l2(y, scale, shift, bn):
    nb = N // bn
    return pl.pallas_call(
        _norm_body,
        grid=(nb,),
        in_specs=[
            pl.BlockSpec((bn, D), lambda b: (b, 0)),
            pl.BlockSpec((1, D), lambda b: (0, 0)),
            pl.BlockSpec((1, D), lambda b: (0, 0)),
        ],
        out_specs=pl.BlockSpec((bn, D), lambda b: (b, 0)),
        out_shape=jax.ShapeDtypeStruct((N, D), jnp.float32),
    )(y, scale, shift)


def kernel(x, edge_index_cites, edge_index_refs, edge_index_likes,
           W_cites, W_refs, W_likes, gamma, beta):
    srcs = jnp.stack(
        [edge_index_cites[0], edge_index_refs[0], edge_index_likes[0]])
    dsts = jnp.stack(
        [edge_index_cites[1], edge_index_refs[1], edge_index_likes[1]])

    idx6 = jnp.concatenate([srcs, dsts]).reshape(2 * NR, NS, NCHD, CHD)
    zeros1 = jnp.zeros((N,), jnp.float32)
    ones_c = jnp.ones((CHD,), jnp.float32)
    deg = _degrees(idx6, zeros1, ones_c)[:, 0, :]
    nsrc = lax.rsqrt(jnp.maximum(deg[:NR], 1.0))
    ndst = lax.rsqrt(jnp.maximum(deg[NR:], 1.0))

    W3 = jnp.stack([W_cites, W_refs, W_likes])
    table = _tables(x, nsrc[:, :, None], W3, 1000)
    t0 = table[0].reshape(NR * N, HALF)
    t1 = table[1].reshape(NR * N, HALF)

    roff = (jnp.arange(NR, dtype=jnp.int32) * N)[:, None]
    pad = PT2 - PT
    src_pad = ((jnp.arange(pad, dtype=jnp.int32) * 97) % N)
    dst_pad = N + (jnp.arange(pad, dtype=jnp.int32) % (NPAD - N))
    srcg3 = (srcs + roff).reshape(NR, NS, PT)
    dsti3 = dsts.reshape(NR, NS, PT)
    srcg = jnp.concatenate(
        [srcg3, jnp.broadcast_to(src_pad, (NR, NS, pad))], axis=2).reshape(-1)
    dsti = jnp.concatenate(
        [dsti3, jnp.broadcast_to(dst_pad, (NR, NS, pad))], axis=2).reshape(-1)
    zrow = jnp.zeros((DR, HALF), jnp.float32)
    agg = _scatter(t0, t1, srcg, dsti, zrow)

    y, st = _combine(agg, ndst[:, :, None], 1000)
    mean = st[0] / N
    var = st[1] / N - mean * mean
    scale = gamma * lax.rsqrt(var + BN_EPS)
    shift = beta - mean * scale

    return _bn_l2(y, scale[None], shift[None], 1000)


# speedup vs baseline: 6.2115x; 1.1448x over previous
import functools

import jax
import jax.numpy as jnp
from jax import lax
from jax.experimental import pallas as pl
from jax.experimental.pallas import tpu as pltpu
from jax.experimental.pallas import tpu_sc as plsc

N = 10000
D = 256
HALF = 128
E = 160000
NR = 3
NCSC = 2
NS = 16
CH = 80
CHD = 80
PT = E // NS
NCHUNK = PT // CH
NCHD = PT // CHD
NPAD = 10240
ROWS_T = NPAD // NS
DR = 80
BN_EPS = 1e-5
L2_EPS = 1e-12

_MESH = dict(core_axis_name="c", subcore_axis_name="s", num_cores=NCSC,
             num_subcores=NS)


def _deg_body(idx6, zeros1, ones_c, deg_out, h0, h1, h2, ibuf, obuf, zbuf,
              hbuf):
    c = lax.axis_index("c")
    s = lax.axis_index("s")
    hists = (h0, h1, h2)
    pltpu.sync_copy(ones_c, obuf)

    @pl.when(s < 3)
    def _():
        pltpu.sync_copy(zeros1, zbuf)

    def _zero(j):
        @pl.when(s == j)
        def _():
            pltpu.sync_copy(zbuf, hists[j])

    for j in range(NR):
        _zero(j)
    plsc.subcore_barrier()

    for j in range(NR):
        a = c * NR + j
        pltpu.sync_copy(idx6.at[a, s], ibuf)

        def _chunk(ci, carry, _hist=hists[j]):
            pltpu.sync_copy(obuf, _hist.at[ibuf.at[ci]], add=True)
            return carry

        lax.fori_loop(0, NCHD, _chunk, 0)
    plsc.subcore_barrier()

    def _drain(j):
        @pl.when(s == j)
        def _():
            pltpu.sync_copy(hists[j], hbuf.at[0])
            pltpu.sync_copy(hbuf, deg_out.at[c * NR + j])

    for j in range(NR):
        _drain(j)


def _degrees(idx6, zeros1, ones_c):
    return pl.kernel(
        _deg_body,
        out_type=jax.ShapeDtypeStruct((2 * NR, 1, N), jnp.float32),
        mesh=plsc.VectorSubcoreMesh(**_MESH),
        scratch_types=[
            pltpu.VMEM_SHARED((N,), jnp.float32),
            pltpu.VMEM_SHARED((N,), jnp.float32),
            pltpu.VMEM_SHARED((N,), jnp.float32),
            pltpu.VMEM((NCHD, CHD), jnp.int32),
            pltpu.VMEM((CHD,), jnp.float32),
            pltpu.VMEM((N,), jnp.float32),
            pltpu.VMEM((1, N), jnp.float32),
        ],
    )(idx6, zeros1, ones_c)


def _mm_body(x_ref, n_ref, w_ref, o_ref):
    xs = x_ref[...] * n_ref[0]
    o_ref[0, 0] = jnp.dot(xs, w_ref[0], preferred_element_type=jnp.float32)


def _tables(x, nsrc_col, W3, bn):
    nb = N // bn
    return pl.pallas_call(
        _mm_body,
        grid=(NCSC, NR, nb),
        in_specs=[
            pl.BlockSpec((bn, D), lambda c, r, b: (b, 0)),
            pl.BlockSpec((1, bn, 1), lambda c, r, b: (r, b, 0)),
            pl.BlockSpec((1, D, HALF), lambda c, r, b: (r, 0, c)),
        ],
        out_specs=pl.BlockSpec((1, 1, bn, HALF), lambda c, r, b: (c, r, b, 0)),
        out_shape=jax.ShapeDtypeStruct((NCSC, NR, N, HALF), jnp.float32),
    )(x, nsrc_col, W3)


def _scat_body(t0, t1, srcg, dsti, zrow, out, acc, sbuf, dbuf, bufs, gsem,
               ssem):
    c = lax.axis_index("c")
    s = lax.axis_index("s")

    def _gather_start(ci, slot):
        idx = sbuf.at[pl.ds(ci * CH, CH)]

        @pl.when(c == 0)
        def _():
            pltpu.async_copy(t0.at[idx], bufs.at[slot], gsem)

        @pl.when(c == 1)
        def _():
            pltpu.async_copy(t1.at[idx], bufs.at[slot], gsem)

    def _gather_wait(ci, slot):
        pltpu.make_async_copy(
            t0.at[sbuf.at[pl.ds(ci * CH, CH)]], bufs.at[slot], gsem).wait()

    def _scatter_wait(ci, slot):
        pltpu.make_async_copy(
            bufs.at[slot], acc.at[dbuf.at[ci]], ssem).wait()

    for r in range(NR):
        pltpu.sync_copy(zrow, bufs.at[0])
        for k in range(ROWS_T // DR):
            pltpu.sync_copy(bufs.at[0], acc.at[pl.ds(s * ROWS_T + k * DR, DR)])
        pltpu.sync_copy(srcg.at[pl.ds((r * NS + s) * PT, PT)], sbuf)
        pltpu.sync_copy(dsti.at[r, s], dbuf)
        _gather_start(0, 0)
        plsc.subcore_barrier()

        def _chunk(ci, carry):
            p = lax.rem(ci, 2)
            pn = lax.rem(ci + 1, 2)

            @pl.when(ci >= 1)
            def _():
                _scatter_wait(ci - 1, pn)

            @pl.when(ci + 1 < NCHUNK)
            def _():
                _gather_start(ci + 1, pn)

            _gather_wait(ci, p)
            pltpu.async_copy(bufs.at[p], acc.at[dbuf.at[ci]], ssem, add=True)
            return carry

        lax.fori_loop(0, NCHUNK, _chunk, 0)
        _scatter_wait(NCHUNK - 1, (NCHUNK - 1) % 2)
        plsc.subcore_barrier()
        for k in range(ROWS_T // DR):
            pltpu.sync_copy(acc.at[pl.ds(s * ROWS_T + k * DR, DR)], bufs.at[0])
            pltpu.sync_copy(
                bufs.at[0], out.at[r, c, pl.ds(s * ROWS_T + k * DR, DR)])


def _scatter(t0, t1, srcg, dsti, zrow):
    return pl.kernel(
        _scat_body,
        out_type=jax.ShapeDtypeStruct((NR, NCSC, NPAD, HALF), jnp.float32),
        mesh=plsc.VectorSubcoreMesh(**_MESH),
        scratch_types=[
            pltpu.VMEM_SHARED((NPAD, HALF), jnp.float32),
            pltpu.VMEM((PT,), jnp.int32),
            pltpu.VMEM((NCHUNK, CH), jnp.int32),
            pltpu.VMEM((2, CH, HALF), jnp.float32),
            pltpu.SemaphoreType.DMA,
            pltpu.SemaphoreType.DMA,
        ],
    )(t0, t1, srcg, dsti, zrow)


def _comb_body(agg_ref, nd_ref, y_ref, st_ref):
    b = pl.program_id(0)
    n0 = nd_ref[0]
    n1 = nd_ref[1]
    n2 = nd_ref[2]
    y0 = agg_ref[0, 0] * n0 + agg_ref[1, 0] * n1 + agg_ref[2, 0] * n2
    y1 = agg_ref[0, 1] * n0 + agg_ref[1, 1] * n1 + agg_ref[2, 1] * n2
    yb = jnp.concatenate([y0, y1], axis=1)
    y_ref[...] = yb
    s1 = jnp.sum(yb, axis=0, keepdims=True)
    s2 = jnp.sum(yb * yb, axis=0, keepdims=True)
    upd = jnp.concatenate(
        [s1, s2, jnp.zeros((6, D), jnp.float32)], axis=0)

    @pl.when(b == 0)
    def _():
        st_ref[...] = jnp.zeros_like(st_ref)

    st_ref[...] += upd


def _combine(agg, nd_col, bn):
    nb = N // bn
    return pl.pallas_call(
        _comb_body,
        grid=(nb,),
        in_specs=[
            pl.BlockSpec((NR, NCSC, bn, HALF), lambda b: (0, 0, b, 0)),
            pl.BlockSpec((NR, bn, 1), lambda b: (0, b, 0)),
        ],
        out_specs=[
            pl.BlockSpec((bn, D), lambda b: (b, 0)),
            pl.BlockSpec((8, D), lambda b: (0, 0)),
        ],
        out_shape=[
            jax.ShapeDtypeStruct((N, D), jnp.float32),
            jax.ShapeDtypeStruct((8, D), jnp.float32),
        ],
    )(agg, nd_col)


def _norm_body(y_ref, sc_ref, sh_ref, o_ref):
    z = y_ref[...] * sc_ref[...] + sh_ref[...]
    nrm = jnp.sqrt(jnp.sum(z * z, axis=1, keepdims=True))
    o_ref[...] = z / jnp.maximum(nrm, L2_EPS)


def _bn_l2(y, scale, shift, bn):
    nb = N // bn
    return pl.pallas_call(
        _norm_body,
        grid=(nb,),
        in_specs=[
            pl.BlockSpec((bn, D), lambda b: (b, 0)),
            pl.BlockSpec((1, D), lambda b: (0, 0)),
            pl.BlockSpec((1, D), lambda b: (0, 0)),
        ],
        out_specs=pl.BlockSpec((bn, D), lambda b: (b, 0)),
        out_shape=jax.ShapeDtypeStruct((N, D), jnp.float32),
    )(y, scale, shift)


def kernel(x, edge_index_cites, edge_index_refs, edge_index_likes,
           W_cites, W_refs, W_likes, gamma, beta):
    srcs = jnp.stack(
        [edge_index_cites[0], edge_index_refs[0], edge_index_likes[0]])
    dsts = jnp.stack(
        [edge_index_cites[1], edge_index_refs[1], edge_index_likes[1]])

    idx6 = jnp.concatenate([srcs, dsts]).reshape(2 * NR, NS, NCHD, CHD)
    zeros1 = jnp.zeros((N,), jnp.float32)
    ones_c = jnp.ones((CHD,), jnp.float32)
    deg = _degrees(idx6, zeros1, ones_c)[:, 0, :]
    nsrc = lax.rsqrt(jnp.maximum(deg[:NR], 1.0))
    ndst = lax.rsqrt(jnp.maximum(deg[NR:], 1.0))

    W3 = jnp.stack([W_cites, W_refs, W_likes])
    table = _tables(x, nsrc[:, :, None], W3, 1000)
    t0 = table[0].reshape(NR * N, HALF)
    t1 = table[1].reshape(NR * N, HALF)

    roff = (jnp.arange(NR, dtype=jnp.int32) * N)[:, None]
    srcg = (srcs + roff).reshape(-1)
    dsti = dsts.reshape(NR, NS, NCHUNK, CH)
    zrow = jnp.zeros((DR, HALF), jnp.float32)
    agg = _scatter(t0, t1, srcg, dsti, zrow)

    y, st = _combine(agg, ndst[:, :, None], 1000)
    mean = st[0] / N
    var = st[1] / N - mean * mean
    scale = gamma * lax.rsqrt(var + BN_EPS)
    shift = beta - mean * scale

    return _bn_l2(y, scale[None], shift[None], 1000)


# speedup vs baseline: 6.7543x; 1.0874x over previous
import functools

import jax
import jax.numpy as jnp
from jax import lax
from jax.experimental import pallas as pl
from jax.experimental.pallas import tpu as pltpu
from jax.experimental.pallas import tpu_sc as plsc

N = 10000
D = 256
HALF = 128
E = 160000
NR = 3
NCSC = 2
NS = 16
CH = 80
CHD = 80
PT = E // NS
NCHUNK = PT // CH
NCHD = PT // CHD
NPAD = 10240
ROWS_T = NPAD // NS
DR = 80
BN_EPS = 1e-5
L2_EPS = 1e-12

_MESH = dict(core_axis_name="c", subcore_axis_name="s", num_cores=NCSC,
             num_subcores=NS)


def _deg_body(idx6, zeros1, ones_c, deg_out, h0, h1, h2, ibuf, obuf, zbuf,
              hbuf):
    c = lax.axis_index("c")
    s = lax.axis_index("s")
    hists = (h0, h1, h2)
    pltpu.sync_copy(ones_c, obuf)

    @pl.when(s < 3)
    def _():
        pltpu.sync_copy(zeros1, zbuf)

    def _zero(j):
        @pl.when(s == j)
        def _():
            pltpu.sync_copy(zbuf, hists[j])

    for j in range(NR):
        _zero(j)
    plsc.subcore_barrier()

    for j in range(NR):
        a = c * NR + j
        pltpu.sync_copy(idx6.at[a, s], ibuf)

        def _chunk(ci, carry, _hist=hists[j]):
            pltpu.sync_copy(obuf, _hist.at[ibuf.at[ci]], add=True)
            return carry

        lax.fori_loop(0, NCHD, _chunk, 0)
    plsc.subcore_barrier()

    def _drain(j):
        @pl.when(s == j)
        def _():
            pltpu.sync_copy(hists[j], hbuf.at[0])
            pltpu.sync_copy(hbuf, deg_out.at[c * NR + j])

    for j in range(NR):
        _drain(j)


def _degrees(idx6, zeros1, ones_c):
    return pl.kernel(
        _deg_body,
        out_type=jax.ShapeDtypeStruct((2 * NR, 1, N), jnp.float32),
        mesh=plsc.VectorSubcoreMesh(**_MESH),
        scratch_types=[
            pltpu.VMEM_SHARED((N,), jnp.float32),
            pltpu.VMEM_SHARED((N,), jnp.float32),
            pltpu.VMEM_SHARED((N,), jnp.float32),
            pltpu.VMEM((NCHD, CHD), jnp.int32),
            pltpu.VMEM((CHD,), jnp.float32),
            pltpu.VMEM((N,), jnp.float32),
            pltpu.VMEM((1, N), jnp.float32),
        ],
    )(idx6, zeros1, ones_c)


def _mm_body(x_ref, n_ref, w_ref, o0_ref, o1_ref):
    xs = x_ref[...] * n_ref[0]
    w = w_ref[0]
    o0_ref[...] = jnp.dot(xs, w[:, :HALF], preferred_element_type=jnp.float32)
    o1_ref[...] = jnp.dot(xs, w[:, HALF:], preferred_element_type=jnp.float32)


def _tables(x, nsrc_col, W3, bn):
    nb = N // bn
    return pl.pallas_call(
        _mm_body,
        grid=(NR, nb),
        in_specs=[
            pl.BlockSpec((bn, D), lambda r, b: (b, 0)),
            pl.BlockSpec((1, bn, 1), lambda r, b: (r, b, 0)),
            pl.BlockSpec((1, D, D), lambda r, b: (r, 0, 0)),
        ],
        out_specs=[
            pl.BlockSpec((bn, HALF), lambda r, b, _nb=nb: (r * _nb + b, 0)),
            pl.BlockSpec((bn, HALF), lambda r, b, _nb=nb: (r * _nb + b, 0)),
        ],
        out_shape=[
            jax.ShapeDtypeStruct((NR * N, HALF), jnp.float32),
            jax.ShapeDtypeStruct((NR * N, HALF), jnp.float32),
        ],
    )(x, nsrc_col, W3)


def _scat_body(t0, t1, srcg, dsti, zrow, out, acc, sbuf, dbuf, bufs, gsem,
               ssem):
    c = lax.axis_index("c")
    s = lax.axis_index("s")

    def _gather_start(ci, slot):
        idx = sbuf.at[pl.ds(ci * CH, CH)]

        @pl.when(c == 0)
        def _():
            pltpu.async_copy(t0.at[idx], bufs.at[slot], gsem)

        @pl.when(c == 1)
        def _():
            pltpu.async_copy(t1.at[idx], bufs.at[slot], gsem)

    def _gather_wait(ci, slot):
        pltpu.make_async_copy(
            t0.at[sbuf.at[pl.ds(ci * CH, CH)]], bufs.at[slot], gsem).wait()

    def _scatter_wait(ci, slot):
        pltpu.make_async_copy(
            bufs.at[slot], acc.at[dbuf.at[ci]], ssem).wait()

    for r in range(NR):
        pltpu.sync_copy(zrow, bufs.at[0])
        for k in range(ROWS_T // DR):
            pltpu.sync_copy(bufs.at[0], acc.at[pl.ds(s * ROWS_T + k * DR, DR)])
        pltpu.sync_copy(srcg.at[pl.ds((r * NS + s) * PT, PT)], sbuf)
        pltpu.sync_copy(dsti.at[r, s], dbuf)
        _gather_start(0, 0)
        plsc.subcore_barrier()

        def _chunk(ci, carry):
            p = lax.rem(ci, 2)
            pn = lax.rem(ci + 1, 2)

            @pl.when(ci >= 1)
            def _():
                _scatter_wait(ci - 1, pn)

            @pl.when(ci + 1 < NCHUNK)
            def _():
                _gather_start(ci + 1, pn)

            _gather_wait(ci, p)
            pltpu.async_copy(bufs.at[p], acc.at[dbuf.at[ci]], ssem, add=True)
            return carry

        lax.fori_loop(0, NCHUNK, _chunk, 0)
        _scatter_wait(NCHUNK - 1, (NCHUNK - 1) % 2)
        plsc.subcore_barrier()
        for k in range(ROWS_T // DR):
            pltpu.sync_copy(acc.at[pl.ds(s * ROWS_T + k * DR, DR)], bufs.at[0])
            pltpu.sync_copy(
                bufs.at[0], out.at[r, c, pl.ds(s * ROWS_T + k * DR, DR)])


def _scatter(t0, t1, srcg, dsti, zrow):
    return pl.kernel(
        _scat_body,
        out_type=jax.ShapeDtypeStruct((NR, NCSC, NPAD, HALF), jnp.float32),
        mesh=plsc.VectorSubcoreMesh(**_MESH),
        scratch_types=[
            pltpu.VMEM_SHARED((NPAD, HALF), jnp.float32),
            pltpu.VMEM((PT,), jnp.int32),
            pltpu.VMEM((NCHUNK, CH), jnp.int32),
            pltpu.VMEM((2, CH, HALF), jnp.float32),
            pltpu.SemaphoreType.DMA,
            pltpu.SemaphoreType.DMA,
        ],
    )(t0, t1, srcg, dsti, zrow)


def _comb_body(agg_ref, nd_ref, y_ref, st_ref):
    b = pl.program_id(0)
    n0 = nd_ref[0]
    n1 = nd_ref[1]
    n2 = nd_ref[2]
    y0 = agg_ref[0, 0] * n0 + agg_ref[1, 0] * n1 + agg_ref[2, 0] * n2
    y1 = agg_ref[0, 1] * n0 + agg_ref[1, 1] * n1 + agg_ref[2, 1] * n2
    yb = jnp.concatenate([y0, y1], axis=1)
    y_ref[...] = yb
    s1 = jnp.sum(yb, axis=0, keepdims=True)
    s2 = jnp.sum(yb * yb, axis=0, keepdims=True)
    upd = jnp.concatenate(
        [s1, s2, jnp.zeros((6, D), jnp.float32)], axis=0)

    @pl.when(b == 0)
    def _():
        st_ref[...] = jnp.zeros_like(st_ref)

    st_ref[...] += upd


def _combine(agg, nd_col, bn):
    nb = N // bn
    return pl.pallas_call(
        _comb_body,
        grid=(nb,),
        in_specs=[
            pl.BlockSpec((NR, NCSC, bn, HALF), lambda b: (0, 0, b, 0)),
            pl.BlockSpec((NR, bn, 1), lambda b: (0, b, 0)),
        ],
        out_specs=[
            pl.BlockSpec((bn, D), lambda b: (b, 0)),
            pl.BlockSpec((8, D), lambda b: (0, 0)),
        ],
        out_shape=[
            jax.ShapeDtypeStruct((N, D), jnp.float32),
            jax.ShapeDtypeStruct((8, D), jnp.float32),
        ],
    )(agg, nd_col)


def _norm_body(y_ref, sc_ref, sh_ref, o_ref):
    z = y_ref[...] * sc_ref[...] + sh_ref[...]
    nrm = jnp.sqrt(jnp.sum(z * z, axis=1, keepdims=True))
    o_ref[...] = z / jnp.maximum(nrm, L2_EPS)


def _bn_l2(y, scale, shift, bn):
    nb = N // bn
    return pl.pallas_call(
        _norm_body,
        grid=(nb,),
        in_specs=[
            pl.BlockSpec((bn, D), lambda b: (b, 0)),
            pl.BlockSpec((1, D), lambda b: (0, 0)),
            pl.BlockSpec((1, D), lambda b: (0, 0)),
        ],
        out_specs=pl.BlockSpec((bn, D), lambda b: (b, 0)),
        out_shape=jax.ShapeDtypeStruct((N, D), jnp.float32),
    )(y, scale, shift)


def kernel(x, edge_index_cites, edge_index_refs, edge_index_likes,
           W_cites, W_refs, W_likes, gamma, beta):
    srcs = jnp.stack(
        [edge_index_cites[0], edge_index_refs[0], edge_index_likes[0]])
    dsts = jnp.stack(
        [edge_index_cites[1], edge_index_refs[1], edge_index_likes[1]])

    idx6 = jnp.concatenate([srcs, dsts]).reshape(2 * NR, NS, NCHD, CHD)
    zeros1 = jnp.zeros((N,), jnp.float32)
    ones_c = jnp.ones((CHD,), jnp.float32)
    deg = _degrees(idx6, zeros1, ones_c)[:, 0, :]
    nsrc = lax.rsqrt(jnp.maximum(deg[:NR], 1.0))
    ndst = lax.rsqrt(jnp.maximum(deg[NR:], 1.0))

    W3 = jnp.stack([W_cites, W_refs, W_likes])
    t0, t1 = _tables(x, nsrc[:, :, None], W3, 1000)

    roff = (jnp.arange(NR, dtype=jnp.int32) * N)[:, None]
    srcg = (srcs + roff).reshape(-1)
    dsti = dsts.reshape(NR, NS, NCHUNK, CH)
    zrow = jnp.zeros((DR, HALF), jnp.float32)
    agg = _scatter(t0, t1, srcg, dsti, zrow)

    y, st = _combine(agg, ndst[:, :, None], 1000)
    mean = st[0] / N
    var = st[1] / N - mean * mean
    scale = gamma * lax.rsqrt(var + BN_EPS)
    shift = beta - mean * scale

    return _bn_l2(y, scale[None], shift[None], 1000)


# speedup vs baseline: 6.8526x; 1.0145x over previous
import functools

import jax
import jax.numpy as jnp
from jax import lax
from jax.experimental import pallas as pl
from jax.experimental.pallas import tpu as pltpu
from jax.experimental.pallas import tpu_sc as plsc

N = 10000
D = 256
HALF = 128
E = 160000
NR = 3
NCSC = 2
NS = 16
CH = 80
CHD = 80
PT = E // NS
NCHUNK = PT // CH
NCHD = PT // CHD
NPAD = 10240
ROWS_T = NPAD // NS
DR = 80
BN_EPS = 1e-5
L2_EPS = 1e-12

_MESH = dict(core_axis_name="c", subcore_axis_name="s", num_cores=NCSC,
             num_subcores=NS)


def _deg_body(idx6, zeros1, ones_c, deg_out, h0, h1, h2, ibuf, obuf, zbuf,
              hbuf):
    c = lax.axis_index("c")
    s = lax.axis_index("s")
    hists = (h0, h1, h2)
    pltpu.sync_copy(ones_c, obuf)

    @pl.when(s < 3)
    def _():
        pltpu.sync_copy(zeros1, zbuf)

    def _zero(j):
        @pl.when(s == j)
        def _():
            pltpu.sync_copy(zbuf, hists[j])

    for j in range(NR):
        _zero(j)
    plsc.subcore_barrier()

    for j in range(NR):
        a = c * NR + j
        pltpu.sync_copy(idx6.at[a, s], ibuf)

        def _chunk(ci, carry, _hist=hists[j]):
            pltpu.sync_copy(obuf, _hist.at[ibuf.at[ci]], add=True)
            return carry

        lax.fori_loop(0, NCHD, _chunk, 0)
    plsc.subcore_barrier()

    def _drain(j):
        @pl.when(s == j)
        def _():
            pltpu.sync_copy(hists[j], hbuf.at[0])
            pltpu.sync_copy(hbuf, deg_out.at[c * NR + j])

    for j in range(NR):
        _drain(j)


def _degrees(idx6, zeros1, ones_c):
    return pl.kernel(
        _deg_body,
        out_type=jax.ShapeDtypeStruct((2 * NR, 1, N), jnp.float32),
        mesh=plsc.VectorSubcoreMesh(**_MESH),
        scratch_types=[
            pltpu.VMEM_SHARED((N,), jnp.float32),
            pltpu.VMEM_SHARED((N,), jnp.float32),
            pltpu.VMEM_SHARED((N,), jnp.float32),
            pltpu.VMEM((NCHD, CHD), jnp.int32),
            pltpu.VMEM((CHD,), jnp.float32),
            pltpu.VMEM((N,), jnp.float32),
            pltpu.VMEM((1, N), jnp.float32),
        ],
    )(idx6, zeros1, ones_c)


def _mmraw_body(x_ref, w_ref, o_ref):
    o_ref[0] = jnp.dot(x_ref[...], w_ref[0],
                       preferred_element_type=jnp.float32)


def _mmraw(x, W3, bn):
    nb = N // bn
    return pl.pallas_call(
        _mmraw_body,
        grid=(NR, nb),
        in_specs=[
            pl.BlockSpec((bn, D), lambda r, b: (b, 0)),
            pl.BlockSpec((1, D, D), lambda r, b: (r, 0, 0)),
        ],
        out_specs=pl.BlockSpec((1, bn, D), lambda r, b: (r, b, 0)),
        out_shape=jax.ShapeDtypeStruct((NR, N, D), jnp.float32),
    )(x, W3)


def _scale_body(g_ref, n_ref, o0_ref, o1_ref):
    gs = g_ref[0] * n_ref[0]
    o0_ref[...] = gs[:, :HALF]
    o1_ref[...] = gs[:, HALF:]


def _scale_split(g, nsrc_col, bn):
    nb = N // bn
    return pl.pallas_call(
        _scale_body,
        grid=(NR, nb),
        in_specs=[
            pl.BlockSpec((1, bn, D), lambda r, b: (r, b, 0)),
            pl.BlockSpec((1, bn, 1), lambda r, b: (r, b, 0)),
        ],
        out_specs=[
            pl.BlockSpec((bn, HALF), lambda r, b, _nb=nb: (r * _nb + b, 0)),
            pl.BlockSpec((bn, HALF), lambda r, b, _nb=nb: (r * _nb + b, 0)),
        ],
        out_shape=[
            jax.ShapeDtypeStruct((NR * N, HALF), jnp.float32),
            jax.ShapeDtypeStruct((NR * N, HALF), jnp.float32),
        ],
    )(g, nsrc_col)


def _scat_body(t0, t1, srcg, dsti, zrow, out, acc, sbuf, dbuf, bufs, gsem,
               ssem):
    c = lax.axis_index("c")
    s = lax.axis_index("s")

    def _gather_start(ci, slot):
        idx = sbuf.at[pl.ds(ci * CH, CH)]

        @pl.when(c == 0)
        def _():
            pltpu.async_copy(t0.at[idx], bufs.at[slot], gsem)

        @pl.when(c == 1)
        def _():
            pltpu.async_copy(t1.at[idx], bufs.at[slot], gsem)

    def _gather_wait(ci, slot):
        pltpu.make_async_copy(
            t0.at[sbuf.at[pl.ds(ci * CH, CH)]], bufs.at[slot], gsem).wait()

    def _scatter_wait(ci, slot):
        pltpu.make_async_copy(
            bufs.at[slot], acc.at[dbuf.at[ci]], ssem).wait()

    for r in range(NR):
        pltpu.sync_copy(zrow, bufs.at[0])
        for k in range(ROWS_T // DR):
            pltpu.sync_copy(bufs.at[0], acc.at[pl.ds(s * ROWS_T + k * DR, DR)])
        pltpu.sync_copy(srcg.at[pl.ds((r * NS + s) * PT, PT)], sbuf)
        pltpu.sync_copy(dsti.at[r, s], dbuf)
        _gather_start(0, 0)
        plsc.subcore_barrier()

        def _chunk(ci, carry):
            p = lax.rem(ci, 2)
            pn = lax.rem(ci + 1, 2)

            @pl.when(ci >= 1)
            def _():
                _scatter_wait(ci - 1, pn)

            @pl.when(ci + 1 < NCHUNK)
            def _():
                _gather_start(ci + 1, pn)

            _gather_wait(ci, p)
            pltpu.async_copy(bufs.at[p], acc.at[dbuf.at[ci]], ssem, add=True)
            return carry

        lax.fori_loop(0, NCHUNK, _chunk, 0)
        _scatter_wait(NCHUNK - 1, (NCHUNK - 1) % 2)
        plsc.subcore_barrier()
        pltpu.sync_copy(acc.at[pl.ds(s * ROWS_T, ROWS_T)],
                        out.at[r, c, pl.ds(s * ROWS_T, ROWS_T)])


def _scatter(t0, t1, srcg, dsti, zrow):
    return pl.kernel(
        _scat_body,
        out_type=jax.ShapeDtypeStruct((NR, NCSC, NPAD, HALF), jnp.float32),
        mesh=plsc.VectorSubcoreMesh(**_MESH),
        scratch_types=[
            pltpu.VMEM_SHARED((NPAD, HALF), jnp.float32),
            pltpu.VMEM((PT,), jnp.int32),
            pltpu.VMEM((NCHUNK, CH), jnp.int32),
            pltpu.VMEM((2, CH, HALF), jnp.float32),
            pltpu.SemaphoreType.DMA,
            pltpu.SemaphoreType.DMA,
        ],
    )(t0, t1, srcg, dsti, zrow)


def _comb_body(agg_ref, nd_ref, y_ref, st_ref):
    b = pl.program_id(0)
    n0 = nd_ref[0]
    n1 = nd_ref[1]
    n2 = nd_ref[2]
    y0 = agg_ref[0, 0] * n0 + agg_ref[1, 0] * n1 + agg_ref[2, 0] * n2
    y1 = agg_ref[0, 1] * n0 + agg_ref[1, 1] * n1 + agg_ref[2, 1] * n2
    yb = jnp.concatenate([y0, y1], axis=1)
    y_ref[...] = yb
    s1 = jnp.sum(yb, axis=0, keepdims=True)
    s2 = jnp.sum(yb * yb, axis=0, keepdims=True)
    upd = jnp.concatenate(
        [s1, s2, jnp.zeros((6, D), jnp.float32)], axis=0)

    @pl.when(b == 0)
    def _():
        st_ref[...] = jnp.zeros_like(st_ref)

    st_ref[...] += upd


def _combine(agg, nd_col, bn):
    nb = N // bn
    return pl.pallas_call(
        _comb_body,
        grid=(nb,),
        in_specs=[
            pl.BlockSpec((NR, NCSC, bn, HALF), lambda b: (0, 0, b, 0)),
            pl.BlockSpec((NR, bn, 1), lambda b: (0, b, 0)),
        ],
        out_specs=[
            pl.BlockSpec((bn, D), lambda b: (b, 0)),
            pl.BlockSpec((8, D), lambda b: (0, 0)),
        ],
        out_shape=[
            jax.ShapeDtypeStruct((N, D), jnp.float32),
            jax.ShapeDtypeStruct((8, D), jnp.float32),
        ],
    )(agg, nd_col)


def _norm_body(y_ref, sc_ref, sh_ref, o_ref):
    z = y_ref[...] * sc_ref[...] + sh_ref[...]
    nrm = jnp.sqrt(jnp.sum(z * z, axis=1, keepdims=True))
    o_ref[...] = z / jnp.maximum(nrm, L2_EPS)


def _bn_l2(y, scale, shift, bn):
    nb = N // bn
    return pl.pallas_call(
        _norm_body,
        grid=(nb,),
        in_specs=[
            pl.BlockSpec((bn, D), lambda b: (b, 0)),
            pl.BlockSpec((1, D), lambda b: (0, 0)),
            pl.BlockSpec((1, D), lambda b: (0, 0)),
        ],
        out_specs=pl.BlockSpec((bn, D), lambda b: (b, 0)),
        out_shape=jax.ShapeDtypeStruct((N, D), jnp.float32),
    )(y, scale, shift)


def kernel(x, edge_index_cites, edge_index_refs, edge_index_likes,
           W_cites, W_refs, W_likes, gamma, beta):
    srcs = jnp.stack(
        [edge_index_cites[0], edge_index_refs[0], edge_index_likes[0]])
    dsts = jnp.stack(
        [edge_index_cites[1], edge_index_refs[1], edge_index_likes[1]])

    W3 = jnp.stack([W_cites, W_refs, W_likes])
    g = _mmraw(x, W3, 1000)

    idx6 = jnp.concatenate([srcs, dsts]).reshape(2 * NR, NS, NCHD, CHD)
    zeros1 = jnp.zeros((N,), jnp.float32)
    ones_c = jnp.ones((CHD,), jnp.float32)
    deg = _degrees(idx6, zeros1, ones_c)[:, 0, :]
    nsrc = lax.rsqrt(jnp.maximum(deg[:NR], 1.0))
    ndst = lax.rsqrt(jnp.maximum(deg[NR:], 1.0))

    t0, t1 = _scale_split(g, nsrc[:, :, None], 1000)

    roff = (jnp.arange(NR, dtype=jnp.int32) * N)[:, None]
    srcg = (srcs + roff).reshape(-1)
    dsti = dsts.reshape(NR, NS, NCHUNK, CH)
    zrow = jnp.zeros((DR, HALF), jnp.float32)
    agg = _scatter(t0, t1, srcg, dsti, zrow)

    y, st = _combine(agg, ndst[:, :, None], 1000)
    mean = st[0] / N
    var = st[1] / N - mean * mean
    scale = gamma * lax.rsqrt(var + BN_EPS)
    shift = beta - mean * scale

    return _bn_l2(y, scale[None], shift[None], 1000)
